# baseline (device time: 225888 ns/iter reference)
import jax
import jax.numpy as jnp
from jax import lax
from jax.experimental import pallas as pl
from jax.experimental.pallas import tpu as pltpu

N_DEV = 16
SQ = 2048
D_MODEL = 1024
H_LOC = 8
DH = 128
QBLK = 256
KWIN = 512
CHUNK = SQ // N_DEV
SCALE = 0.08838834764831843


def kernel(x, Wq, K_ext, V_ext, Wo):
    my = lax.axis_index("i")
    bf = jnp.bfloat16
    wq_l = lax.dynamic_slice(Wq, (0, my * (H_LOC * DH)), (D_MODEL, H_LOC * DH))
    wo_l = lax.dynamic_slice(Wo, (my * (H_LOC * DH), 0), (H_LOC * DH, D_MODEL))
    xs = x[0].astype(bf)
    k = K_ext[0].astype(bf)
    v = V_ext[0].astype(bf)

    def body(x_ref, wq_ref, k_ref, v_ref, wo_ref, out_ref,
             acc_ref, recv_ref, send_sems, recv_sems):
        me = lax.axis_index("i")
        left = lax.rem(me + N_DEV - 1, N_DEV)
        right = lax.rem(me + 1, N_DEV)

        barrier = pltpu.get_barrier_semaphore()
        for nbr in (left, right):
            pl.semaphore_signal(barrier, inc=1, device_id=(nbr,),
                                device_id_type=pl.DeviceIdType.MESH)
        pl.semaphore_wait(barrier, 2)

        def compute_block(b, carry):
            qs = b * QBLK
            ks = jnp.clip(qs - 128, 0, SQ - KWIN)
            qblk = lax.dot_general(
                x_ref[pl.ds(qs, QBLK), :], wq_ref[:, :],
                (((1,), (0,)), ((), ())),
                preferred_element_type=jnp.float32)
            qblk = (qblk * SCALE).astype(jnp.bfloat16)
            qi = qs + lax.broadcasted_iota(jnp.int32, (QBLK, KWIN), 0)
            ki = ks + lax.broadcasted_iota(jnp.int32, (QBLK, KWIN), 1)
            mask = jnp.abs(qi - ki) <= 128
            pieces = []
            for h in range(H_LOC):
                qh = qblk[:, h * DH:(h + 1) * DH]
                kh = k_ref[pl.ds(ks, KWIN), h, :]
                s = lax.dot_general(qh, kh, (((1,), (1,)), ((), ())),
                                    preferred_element_type=jnp.float32)
                s = jnp.where(mask, s, -1e9)
                m = jnp.max(s, axis=1, keepdims=True)
                w = jnp.exp(s - m)
                w = w / jnp.sum(w, axis=1, keepdims=True)
                vh = v_ref[pl.ds(ks, KWIN), h, :]
                ctx = lax.dot_general(w.astype(jnp.bfloat16), vh,
                                      (((1,), (0,)), ((), ())),
                                      preferred_element_type=jnp.float32)
                pieces.append(ctx.astype(jnp.bfloat16))
            ctxb = jnp.concatenate(pieces, axis=1)
            pb = lax.dot_general(ctxb, wo_ref[:, :],
                                 (((1,), (0,)), ((), ())),
                                 preferred_element_type=jnp.float32)
            acc_ref[pl.ds(qs, QBLK), :] = pb.astype(jnp.bfloat16)
            return carry

        lax.fori_loop(0, SQ // QBLK, compute_block, 0)

        for s in range(N_DEV - 1):
            row_s = lax.rem(me - s + 2 * N_DEV, N_DEV) * CHUNK
            rdma = pltpu.make_async_remote_copy(
                src_ref=acc_ref.at[pl.ds(row_s, CHUNK), :],
                dst_ref=recv_ref.at[s],
                send_sem=send_sems.at[s],
                recv_sem=recv_sems.at[s],
                device_id=(right,),
                device_id_type=pl.DeviceIdType.MESH)
            rdma.start()
            rdma.wait()
            row_r = lax.rem(me - s - 1 + 2 * N_DEV, N_DEV) * CHUNK
            acc_ref[pl.ds(row_r, CHUNK), :] = (
                acc_ref[pl.ds(row_r, CHUNK), :].astype(jnp.float32)
                + recv_ref[s].astype(jnp.float32)).astype(jnp.bfloat16)

        for s in range(N_DEV - 1):
            g = (N_DEV - 1) + s
            row_s = lax.rem(me + 1 - s + 2 * N_DEV, N_DEV) * CHUNK
            rdma = pltpu.make_async_remote_copy(
                src_ref=acc_ref.at[pl.ds(row_s, CHUNK), :],
                dst_ref=recv_ref.at[g],
                send_sem=send_sems.at[g],
                recv_sem=recv_sems.at[g],
                device_id=(right,),
                device_id_type=pl.DeviceIdType.MESH)
            rdma.start()
            rdma.wait()
            row_r = lax.rem(me - s + 2 * N_DEV, N_DEV) * CHUNK
            acc_ref[pl.ds(row_r, CHUNK), :] = recv_ref[g]

        out_ref[0, :, :] = acc_ref[:, :]

    n_slots = 2 * (N_DEV - 1)
    return pl.pallas_call(
        body,
        out_shape=jax.ShapeDtypeStruct((1, SQ, D_MODEL), jnp.bfloat16),
        in_specs=[pl.BlockSpec(memory_space=pltpu.VMEM)] * 5,
        out_specs=pl.BlockSpec(memory_space=pltpu.VMEM),
        scratch_shapes=[
            pltpu.VMEM((SQ, D_MODEL), jnp.bfloat16),
            pltpu.VMEM((n_slots, CHUNK, D_MODEL), jnp.bfloat16),
            pltpu.SemaphoreType.DMA((n_slots,)),
            pltpu.SemaphoreType.DMA((n_slots,)),
        ],
        compiler_params=pltpu.CompilerParams(collective_id=0),
    )(xs, wq_l.astype(bf), k, v, wo_l.astype(bf))


# device time: 176109 ns/iter; 1.2827x vs baseline; 1.2827x over previous
import jax
import jax.numpy as jnp
from jax import lax
from jax.experimental import pallas as pl
from jax.experimental.pallas import tpu as pltpu

N_DEV = 16
SQ = 2048
D_MODEL = 1024
H_LOC = 8
DH = 128
QBLK = 256
KWIN = 512
CHUNK = SQ // N_DEV
LANES = 4
SUB = CHUNK // LANES
N_STEP = N_DEV - 1
SCALE = 0.08838834764831843


def _ring_pos(m):
    z = m // 4
    w = lax.rem(m, 4)
    up = lax.rem(w, 2) == 0
    return 4 * w + jnp.where(up, z, 3 - z)


def _mesh_of_pos(r):
    rr = lax.rem(r + 2 * N_DEV, N_DEV)
    w = rr // 4
    i = lax.rem(rr, 4)
    z = jnp.where(lax.rem(w, 2) == 0, i, 3 - i)
    return 4 * z + w


def kernel(x, Wq, K_ext, V_ext, Wo):
    my = lax.axis_index("i")
    bf = jnp.bfloat16
    wq_l = lax.dynamic_slice(Wq, (0, my * (H_LOC * DH)), (D_MODEL, H_LOC * DH))
    wo_l = lax.dynamic_slice(Wo, (my * (H_LOC * DH), 0), (H_LOC * DH, D_MODEL))
    xs = x[0].astype(bf)
    k = K_ext[0].astype(bf)
    v = V_ext[0].astype(bf)

    def body(x_ref, wq_ref, k_ref, v_ref, wo_ref, out_ref,
             acc_ref, recv_ref, send_sems, recv_sems):
        me = lax.axis_index("i")
        r = _ring_pos(me)
        left = _mesh_of_pos(r - 1)
        right = _mesh_of_pos(r + 1)

        barrier = pltpu.get_barrier_semaphore()
        for nbr in (left, right):
            pl.semaphore_signal(barrier, inc=1, device_id=(nbr,),
                                device_id_type=pl.DeviceIdType.MESH)
        pl.semaphore_wait(barrier, 2)

        def compute_block(b, carry):
            qs = b * QBLK
            ks = jnp.clip(qs - 128, 0, SQ - KWIN)
            qblk = lax.dot_general(
                x_ref[pl.ds(qs, QBLK), :], wq_ref[:, :],
                (((1,), (0,)), ((), ())),
                preferred_element_type=jnp.float32)
            qblk = (qblk * SCALE).astype(jnp.bfloat16)
            qi = qs + lax.broadcasted_iota(jnp.int32, (QBLK, KWIN), 0)
            ki = ks + lax.broadcasted_iota(jnp.int32, (QBLK, KWIN), 1)
            mask = jnp.abs(qi - ki) <= 128
            pieces = []
            for h in range(H_LOC):
                qh = qblk[:, h * DH:(h + 1) * DH]
                kh = k_ref[pl.ds(ks, KWIN), h, :]
                s = lax.dot_general(qh, kh, (((1,), (1,)), ((), ())),
                                    preferred_element_type=jnp.float32)
                s = jnp.where(mask, s, -1e9)
                m = jnp.max(s, axis=1, keepdims=True)
                w = jnp.exp(s - m)
                w = w / jnp.sum(w, axis=1, keepdims=True)
                vh = v_ref[pl.ds(ks, KWIN), h, :]
                ctx = lax.dot_general(w.astype(jnp.bfloat16), vh,
                                      (((1,), (0,)), ((), ())),
                                      preferred_element_type=jnp.float32)
                pieces.append(ctx.astype(jnp.bfloat16))
            ctxb = jnp.concatenate(pieces, axis=1)
            pb = lax.dot_general(ctxb, wo_ref[:, :],
                                 (((1,), (0,)), ((), ())),
                                 preferred_element_type=jnp.float32)
            acc_ref[pl.ds(qs, QBLK), :] = pb.astype(jnp.bfloat16)
            return carry

        lax.fori_loop(0, SQ // QBLK, compute_block, 0)

        def sub_rows(chunk_pos, g):
            return lax.rem(chunk_pos + 2 * N_DEV, N_DEV) * CHUNK + g * SUB

        def send(slot, row, g):
            rdma = pltpu.make_async_remote_copy(
                src_ref=acc_ref.at[pl.ds(row, SUB), :],
                dst_ref=recv_ref.at[slot],
                send_sem=send_sems.at[slot],
                recv_sem=recv_sems.at[slot],
                device_id=(right,),
                device_id_type=pl.DeviceIdType.MESH)
            rdma.start()

        def recv_wait(slot):
            rdma = pltpu.make_async_remote_copy(
                src_ref=acc_ref.at[pl.ds(0, SUB), :],
                dst_ref=recv_ref.at[slot],
                send_sem=send_sems.at[slot],
                recv_sem=recv_sems.at[slot],
                device_id=(right,),
                device_id_type=pl.DeviceIdType.MESH)
            rdma.wait_recv()

        for g in range(LANES):
            send(g, sub_rows(r, g), g)

        def rs_step(s, carry):
            for g in range(LANES):
                slot = s * LANES + g
                recv_wait(slot)
                row = sub_rows(r - s - 1, g)
                acc_ref[pl.ds(row, SUB), :] = (
                    acc_ref[pl.ds(row, SUB), :].astype(jnp.float32)
                    + recv_ref[slot].astype(jnp.float32)
                ).astype(jnp.bfloat16)

                @pl.when(s < N_STEP - 1)
                def _():
                    send(slot + LANES, row, g)
            return carry

        lax.fori_loop(0, N_STEP, rs_step, 0)

        ag0 = N_STEP * LANES
        for g in range(LANES):
            send(ag0 + g, sub_rows(r + 1, g), g)

        def ag_step(s, carry):
            for g in range(LANES):
                slot = ag0 + s * LANES + g
                recv_wait(slot)

                @pl.when(s < N_STEP - 1)
                def _():
                    rdma = pltpu.make_async_remote_copy(
                        src_ref=recv_ref.at[slot],
                        dst_ref=recv_ref.at[slot + LANES],
                        send_sem=send_sems.at[slot + LANES],
                        recv_sem=recv_sems.at[slot + LANES],
                        device_id=(right,),
                        device_id_type=pl.DeviceIdType.MESH)
                    rdma.start()

                row = sub_rows(r - s, g)
                acc_ref[pl.ds(row, SUB), :] = recv_ref[slot]
            return carry

        lax.fori_loop(0, N_STEP, ag_step, 0)

        def drain(i, carry):
            rdma = pltpu.make_async_remote_copy(
                src_ref=acc_ref.at[pl.ds(0, SUB), :],
                dst_ref=recv_ref.at[i],
                send_sem=send_sems.at[i],
                recv_sem=recv_sems.at[i],
                device_id=(right,),
                device_id_type=pl.DeviceIdType.MESH)
            rdma.wait_send()
            return carry

        lax.fori_loop(0, 2 * N_STEP * LANES, drain, 0)

        out_ref[0, :, :] = acc_ref[:, :]

    n_slots = 2 * N_STEP * LANES
    return pl.pallas_call(
        body,
        out_shape=jax.ShapeDtypeStruct((1, SQ, D_MODEL), jnp.bfloat16),
        in_specs=[pl.BlockSpec(memory_space=pltpu.VMEM)] * 5,
        out_specs=pl.BlockSpec(memory_space=pltpu.VMEM),
        scratch_shapes=[
            pltpu.VMEM((SQ, D_MODEL), jnp.bfloat16),
            pltpu.VMEM((n_slots, SUB, D_MODEL), jnp.bfloat16),
            pltpu.SemaphoreType.DMA((n_slots,)),
            pltpu.SemaphoreType.DMA((n_slots,)),
        ],
        compiler_params=pltpu.CompilerParams(collective_id=0),
    )(xs, wq_l.astype(bf), k, v, wo_l.astype(bf))


# device time: 175498 ns/iter; 1.2871x vs baseline; 1.0035x over previous
import jax
import jax.numpy as jnp
from jax import lax
from jax.experimental import pallas as pl
from jax.experimental.pallas import tpu as pltpu

N_DEV = 16
SQ = 2048
D_MODEL = 1024
H_LOC = 8
DH = 128
QBLK = 256
KWIN = 512
CHUNK = SQ // N_DEV
LANES = 4
SUB = CHUNK // LANES
N_STEP = N_DEV - 1
SCALE = 0.08838834764831843


def _ring_pos(m):
    z = m // 4
    w = lax.rem(m, 4)
    up = lax.rem(w, 2) == 0
    return 4 * w + jnp.where(up, z, 3 - z)


def _mesh_of_pos(r):
    rr = lax.rem(r + 2 * N_DEV, N_DEV)
    w = rr // 4
    i = lax.rem(rr, 4)
    z = jnp.where(lax.rem(w, 2) == 0, i, 3 - i)
    return 4 * z + w


def kernel(x, Wq, K_ext, V_ext, Wo):
    my = lax.axis_index("i")
    bf = jnp.bfloat16
    wq_l = lax.dynamic_slice(Wq, (0, my * (H_LOC * DH)), (D_MODEL, H_LOC * DH))
    wo_l = lax.dynamic_slice(Wo, (my * (H_LOC * DH), 0), (H_LOC * DH, D_MODEL))
    xs = x[0].astype(bf)
    k = K_ext[0].astype(bf)
    v = V_ext[0].astype(bf)

    def body(x_ref, wq_ref, k_ref, v_ref, wo_ref, out_ref,
             acc_ref, recv_ref, send_sems, recv_sems):
        me = lax.axis_index("i")
        r = _ring_pos(me)
        left = _mesh_of_pos(r - 1)
        right = _mesh_of_pos(r + 1)

        barrier = pltpu.get_barrier_semaphore()
        for nbr in (left, right):
            pl.semaphore_signal(barrier, inc=1, device_id=(nbr,),
                                device_id_type=pl.DeviceIdType.MESH)
        pl.semaphore_wait(barrier, 2)

        def compute_block(b, carry):
            qs = b * QBLK
            ks = jnp.clip(qs - 128, 0, SQ - KWIN)
            qblk = lax.dot_general(
                x_ref[pl.ds(qs, QBLK), :], wq_ref[:, :],
                (((1,), (0,)), ((), ())),
                preferred_element_type=jnp.float32)
            qblk = (qblk * SCALE).astype(jnp.bfloat16)
            qi = qs + lax.broadcasted_iota(jnp.int32, (QBLK, KWIN), 0)
            ki = ks + lax.broadcasted_iota(jnp.int32, (QBLK, KWIN), 1)
            mask = jnp.abs(qi - ki) <= 128
            pieces = []
            for h in range(H_LOC):
                qh = qblk[:, h * DH:(h + 1) * DH]
                kh = k_ref[pl.ds(ks, KWIN), h, :]
                s = lax.dot_general(qh, kh, (((1,), (1,)), ((), ())),
                                    preferred_element_type=jnp.float32)
                s = jnp.where(mask, s, -1e9)
                m = jnp.max(s, axis=1, keepdims=True)
                w = jnp.exp(s - m)
                w = w / jnp.sum(w, axis=1, keepdims=True)
                vh = v_ref[pl.ds(ks, KWIN), h, :]
                ctx = lax.dot_general(w.astype(jnp.bfloat16), vh,
                                      (((1,), (0,)), ((), ())),
                                      preferred_element_type=jnp.float32)
                pieces.append(ctx.astype(jnp.bfloat16))
            ctxb = jnp.concatenate(pieces, axis=1)
            pb = lax.dot_general(ctxb, wo_ref[:, :],
                                 (((1,), (0,)), ((), ())),
                                 preferred_element_type=jnp.float32)
            acc_ref[pl.ds(qs, QBLK), :] = pb.astype(jnp.bfloat16)
            return carry

        lax.fori_loop(0, SQ // QBLK, compute_block, 0)

        def sub_rows(chunk_pos, g):
            return lax.rem(chunk_pos + 2 * N_DEV, N_DEV) * CHUNK + g * SUB

        def rs_send(slot, row):
            rdma = pltpu.make_async_remote_copy(
                src_ref=acc_ref.at[pl.ds(row, SUB), :],
                dst_ref=recv_ref.at[slot],
                send_sem=send_sems.at[slot],
                recv_sem=recv_sems.at[slot],
                device_id=(right,),
                device_id_type=pl.DeviceIdType.MESH)
            rdma.start()
            return rdma

        def rs_recv_wait(slot):
            rdma = pltpu.make_async_remote_copy(
                src_ref=acc_ref.at[pl.ds(0, SUB), :],
                dst_ref=recv_ref.at[slot],
                send_sem=send_sems.at[slot],
                recv_sem=recv_sems.at[slot],
                device_id=(right,),
                device_id_type=pl.DeviceIdType.MESH)
            rdma.wait_recv()

        def ag_send(slot, row):
            rdma = pltpu.make_async_remote_copy(
                src_ref=acc_ref.at[pl.ds(row, SUB), :],
                dst_ref=acc_ref.at[pl.ds(row, SUB), :],
                send_sem=send_sems.at[slot],
                recv_sem=recv_sems.at[slot],
                device_id=(right,),
                device_id_type=pl.DeviceIdType.MESH)
            rdma.start()
            return rdma

        def ag_recv_wait(slot, row):
            rdma = pltpu.make_async_remote_copy(
                src_ref=acc_ref.at[pl.ds(row, SUB), :],
                dst_ref=acc_ref.at[pl.ds(row, SUB), :],
                send_sem=send_sems.at[slot],
                recv_sem=recv_sems.at[slot],
                device_id=(right,),
                device_id_type=pl.DeviceIdType.MESH)
            rdma.wait_recv()

        sent = {}

        def roll_wait(slot):
            sent.pop(slot).wait_send()

        for g in range(LANES):
            sent[g] = rs_send(g, sub_rows(r, g))

        for s in range(N_STEP):
            for g in range(LANES):
                slot = s * LANES + g
                rs_recv_wait(slot)
                row = sub_rows(r - s - 1, g)
                acc_ref[pl.ds(row, SUB), :] = (
                    acc_ref[pl.ds(row, SUB), :] + recv_ref[slot])
                if s < N_STEP - 1:
                    sent[slot + LANES] = rs_send(slot + LANES, row)
                if s >= 2:
                    roll_wait(slot - 2 * LANES)
        for g in range(LANES):
            roll_wait((N_STEP - 2) * LANES + g)
            roll_wait((N_STEP - 1) * LANES + g)

        ag0 = N_STEP * LANES
        for g in range(LANES):
            sent[ag0 + g] = ag_send(ag0 + g, sub_rows(r + 1, g))

        for s in range(N_STEP):
            for g in range(LANES):
                slot = ag0 + s * LANES + g
                row = sub_rows(r - s, g)
                ag_recv_wait(slot, row)
                if s < N_STEP - 1:
                    sent[slot + LANES] = ag_send(slot + LANES, row)
                if s >= 2:
                    roll_wait(slot - 2 * LANES)
        for g in range(LANES):
            roll_wait(ag0 + (N_STEP - 2) * LANES + g)
            roll_wait(ag0 + (N_STEP - 1) * LANES + g)

        out_ref[0, :, :] = acc_ref[:, :]

    n_slots = 2 * N_STEP * LANES
    return pl.pallas_call(
        body,
        out_shape=jax.ShapeDtypeStruct((1, SQ, D_MODEL), jnp.bfloat16),
        in_specs=[pl.BlockSpec(memory_space=pltpu.VMEM)] * 5,
        out_specs=pl.BlockSpec(memory_space=pltpu.VMEM),
        scratch_shapes=[
            pltpu.VMEM((SQ, D_MODEL), jnp.bfloat16),
            pltpu.VMEM((N_STEP * LANES, SUB, D_MODEL), jnp.bfloat16),
            pltpu.SemaphoreType.DMA((n_slots,)),
            pltpu.SemaphoreType.DMA((n_slots,)),
        ],
        compiler_params=pltpu.CompilerParams(collective_id=0),
    )(xs, wq_l.astype(bf), k, v, wo_l.astype(bf))
